# baseline (speedup 1.0000x reference)
import functools

import jax
import jax.numpy as jnp
from jax import lax
from jax.experimental import pallas as pl
from jax.experimental.pallas import tpu as pltpu
from jax.experimental.pallas import tpu_sc as plsc

_VOCAB = 100000
_DIM = 768
_B = 4
_S = 2048
_N_IDX = _B * _S
_NC = 2
_NS = 16
_NW = _NC * _NS
_PER_W = _N_IDX // _NW
_CHUNK = 32
_NCHUNK = _PER_W // _CHUNK
_GROUPS = _DIM // 32


def _make_kernel():
  mesh = plsc.VectorSubcoreMesh(
      core_axis_name="c", subcore_axis_name="s",
      num_cores=_NC, num_subcores=_NS)

  @functools.partial(
      pl.kernel,
      out_type=[jax.ShapeDtypeStruct((_N_IDX, _DIM), jnp.bfloat16)] * 3,
      mesh=mesh,
      compiler_params=pltpu.CompilerParams(
          use_tc_tiling_on_sc=False, needs_layout_passes=False),
      scratch_types=[
          pltpu.VMEM((_PER_W,), jnp.int32),
          pltpu.VMEM((_CHUNK, _DIM), jnp.float32),
          pltpu.VMEM((_CHUNK, _DIM), jnp.float32),
          pltpu.VMEM((_CHUNK, _DIM), jnp.bfloat16),
          pltpu.VMEM((_CHUNK, _DIM), jnp.bfloat16),
          pltpu.SemaphoreType.DMA,
          pltpu.SemaphoreType.DMA,
          pltpu.SemaphoreType.DMA,
          pltpu.SemaphoreType.DMA,
      ],
  )
  def emb_kernel(idx_hbm, w0, w1, w2, o0, o1, o2,
                 idx_v, in0, in1, ob0, ob1, gs0, gs1, ss0, ss1):
    wid = lax.axis_index("s") * _NC + lax.axis_index("c")
    base = wid * _PER_W
    pltpu.sync_copy(idx_hbm.at[pl.ds(base, _PER_W)], idx_v)

    tables = (w0, w1, w2)
    outs = (o0, o1, o2)
    inbufs = (in0, in1)
    outbufs = (ob0, ob1)
    gsems = (gs0, gs1)
    ssems = (ss0, ss1)

    two_iota = lax.iota(jnp.int32, 16) * 2

    def start_gather(step):
      t, c = divmod(step, _NCHUNK)
      return pltpu.async_copy(
          tables[t].at[idx_v.at[pl.ds(c * _CHUNK, _CHUNK)]],
          inbufs[step % 2], gsems[step % 2])

    def convert(inb, outb):
      @plsc.parallel_loop(0, _CHUNK)
      def _rows(r):
        rr = jnp.full((16,), r, jnp.int32)

        @plsc.parallel_loop(0, _GROUPS, unroll=4)
        def _groups(k):
          col_e = k * 32 + two_iota
          a = plsc.load_gather(inb, [rr, col_e])
          b = plsc.load_gather(inb, [rr, col_e + 1])
          outb[r, pl.ds(k * 32, 32)] = plsc.pack(
              a, b, format=plsc.PackFormat.INTERLEAVED)

    n_steps = 3 * _NCHUNK
    gcopies = [None, None]
    scopies = [None, None]
    gcopies[0] = start_gather(0)
    for step in range(n_steps):
      p = step % 2
      if step + 1 < n_steps:
        gcopies[(step + 1) % 2] = start_gather(step + 1)
      gcopies[p].wait()
      if scopies[p] is not None:
        scopies[p].wait()
      convert(inbufs[p], outbufs[p])
      t, c = divmod(step, _NCHUNK)
      scopies[p] = pltpu.async_copy(
          outbufs[p], outs[t].at[pl.ds(base + c * _CHUNK, _CHUNK)], ssems[p])
    scopies[0].wait()
    scopies[1].wait()

  return emb_kernel


_emb_kernel = _make_kernel()


def kernel(inputs, W0, W1, W2):
  idx = inputs.reshape(_N_IDX)
  e0, e1, e2 = _emb_kernel(idx, W0, W1, W2)
  e0 = e0.reshape(_B, _S, _DIM)
  e1 = e1.reshape(_B, _S, _DIM)
  e2 = e2.reshape(_B, _S, _DIM)
  return (e0, e1, e2, e0, e1, e2)


# speedup vs baseline: 12.2488x; 12.2488x over previous
import functools

import jax
import jax.numpy as jnp
from jax import lax
from jax.experimental import pallas as pl
from jax.experimental.pallas import tpu as pltpu
from jax.experimental.pallas import tpu_sc as plsc

_VOCAB = 100000
_DIM = 768
_B = 4
_S = 2048
_N_IDX = _B * _S
_NC = 2
_NS = 16
_NW = _NC * _NS
_PER_W = _N_IDX // _NW
_CHUNK = 32
_NCHUNK = _PER_W // _CHUNK
_GROUPS = _DIM // 32


def _make_kernel():
  mesh = plsc.VectorSubcoreMesh(
      core_axis_name="c", subcore_axis_name="s",
      num_cores=_NC, num_subcores=_NS)

  @functools.partial(
      pl.kernel,
      out_type=[jax.ShapeDtypeStruct((_N_IDX, _DIM), jnp.bfloat16)] * 3,
      mesh=mesh,
      compiler_params=pltpu.CompilerParams(needs_layout_passes=False),
      scratch_types=[
          pltpu.VMEM((_PER_W,), jnp.int32),
          pltpu.VMEM((_CHUNK, _DIM), jnp.float32),
          pltpu.VMEM((_CHUNK, _DIM), jnp.float32),
          pltpu.VMEM((_CHUNK, _DIM), jnp.bfloat16),
          pltpu.VMEM((_CHUNK, _DIM), jnp.bfloat16),
          pltpu.SemaphoreType.DMA,
          pltpu.SemaphoreType.DMA,
          pltpu.SemaphoreType.DMA,
          pltpu.SemaphoreType.DMA,
      ],
  )
  def emb_kernel(idx_hbm, w0, w1, w2, o0, o1, o2,
                 idx_v, in0, in1, ob0, ob1, gs0, gs1, ss0, ss1):
    wid = lax.axis_index("s") * _NC + lax.axis_index("c")
    base = wid * _PER_W
    pltpu.sync_copy(idx_hbm.at[pl.ds(base, _PER_W)], idx_v)

    tables = (w0, w1, w2)
    outs = (o0, o1, o2)
    inbufs = (in0, in1)
    outbufs = (ob0, ob1)
    gsems = (gs0, gs1)
    ssems = (ss0, ss1)

    two_iota = lax.iota(jnp.int32, 16) * 2

    def start_gather(step):
      t, c = divmod(step, _NCHUNK)
      return pltpu.async_copy(
          tables[t].at[idx_v.at[pl.ds(c * _CHUNK, _CHUNK)]],
          inbufs[step % 2], gsems[step % 2])

    def convert(inb, outb):
      @plsc.parallel_loop(0, _CHUNK)
      def _rows(r):
        rr = jnp.full((16,), r, jnp.int32)

        @plsc.parallel_loop(0, _GROUPS, unroll=4)
        def _groups(k):
          col_e = k * 32 + two_iota
          a = plsc.load_gather(inb, [rr, col_e])
          b = plsc.load_gather(inb, [rr, col_e + 1])
          outb[r, pl.ds(k * 32, 32)] = plsc.pack(
              a, b, format=plsc.PackFormat.INTERLEAVED)

    n_steps = 3 * _NCHUNK
    gcopies = [None, None]
    scopies = [None, None]
    gcopies[0] = start_gather(0)
    for step in range(n_steps):
      p = step % 2
      if step + 1 < n_steps:
        gcopies[(step + 1) % 2] = start_gather(step + 1)
      gcopies[p].wait()
      if scopies[p] is not None:
        scopies[p].wait()
      convert(inbufs[p], outbufs[p])
      t, c = divmod(step, _NCHUNK)
      scopies[p] = pltpu.async_copy(
          outbufs[p], outs[t].at[pl.ds(base + c * _CHUNK, _CHUNK)], ssems[p])
    scopies[0].wait()
    scopies[1].wait()

  return emb_kernel


_emb_kernel = _make_kernel()


def kernel(inputs, W0, W1, W2):
  idx = inputs.reshape(_N_IDX)
  e0, e1, e2 = _emb_kernel(idx, W0, W1, W2)
  e0 = e0.reshape(_B, _S, _DIM)
  e1 = e1.reshape(_B, _S, _DIM)
  e2 = e2.reshape(_B, _S, _DIM)
  return (e0, e1, e2, e0, e1, e2)


# speedup vs baseline: 14.3698x; 1.1732x over previous
import functools

import jax
import jax.numpy as jnp
from jax import lax
from jax.experimental import pallas as pl
from jax.experimental.pallas import tpu as pltpu
from jax.experimental.pallas import tpu_sc as plsc

_VOCAB = 100000
_DIM = 768
_B = 4
_S = 2048
_N_IDX = _B * _S
_NC = 2
_NS = 16
_NW = _NC * _NS
_PER_W = _N_IDX // _NW
_CHUNK = 32
_NCHUNK = _PER_W // _CHUNK
_GROUPS = _DIM // 32


def _make_kernel():
  mesh = plsc.VectorSubcoreMesh(
      core_axis_name="c", subcore_axis_name="s",
      num_cores=_NC, num_subcores=_NS)

  @functools.partial(
      pl.kernel,
      out_type=[jax.ShapeDtypeStruct((_N_IDX, _DIM), jnp.bfloat16)] * 6,
      mesh=mesh,
      compiler_params=pltpu.CompilerParams(needs_layout_passes=False),
      scratch_types=[
          pltpu.VMEM((_PER_W,), jnp.int32),
          pltpu.VMEM((_CHUNK, _DIM), jnp.float32),
          pltpu.VMEM((_CHUNK, _DIM), jnp.float32),
          pltpu.VMEM((_CHUNK, _DIM), jnp.bfloat16),
          pltpu.VMEM((_CHUNK, _DIM), jnp.bfloat16),
          pltpu.SemaphoreType.DMA,
          pltpu.SemaphoreType.DMA,
          pltpu.SemaphoreType.DMA,
          pltpu.SemaphoreType.DMA,
      ],
  )
  def emb_kernel(idx_hbm, w0, w1, w2, o0, o1, o2, o3, o4, o5,
                 idx_v, in0, in1, ob0, ob1, gs0, gs1, ss0, ss1):
    wid = lax.axis_index("s") * _NC + lax.axis_index("c")
    base = wid * _PER_W
    row = base // _S
    col = base % _S
    pltpu.sync_copy(idx_hbm.at[row, pl.ds(col, _PER_W)], idx_v)

    tables = (w0, w1, w2)
    outs = ((o0, o3), (o1, o4), (o2, o5))
    inbufs = (in0, in1)
    outbufs = (ob0, ob1)
    gsems = (gs0, gs1)
    ssems = (ss0, ss1)

    two_iota = lax.iota(jnp.int32, 16) * 2

    def start_gather(step):
      t, c = divmod(step, _NCHUNK)
      return pltpu.async_copy(
          tables[t].at[idx_v.at[pl.ds(c * _CHUNK, _CHUNK)]],
          inbufs[step % 2], gsems[step % 2])

    def convert(inb, outb):
      @plsc.parallel_loop(0, _CHUNK)
      def _rows(r):
        rr = jnp.full((16,), r, jnp.int32)

        @plsc.parallel_loop(0, _GROUPS, unroll=4)
        def _groups(k):
          col_e = k * 32 + two_iota
          a = plsc.load_gather(inb, [rr, col_e])
          b = plsc.load_gather(inb, [rr, col_e + 1])
          outb[r, pl.ds(k * 32, 32)] = plsc.pack(
              a, b, format=plsc.PackFormat.INTERLEAVED)

    n_steps = 3 * _NCHUNK
    gcopies = [None, None]
    scopies = [None, None]
    gcopies[0] = start_gather(0)
    for step in range(n_steps):
      p = step % 2
      if step + 1 < n_steps:
        gcopies[(step + 1) % 2] = start_gather(step + 1)
      gcopies[p].wait()
      if scopies[p] is not None:
        for cp in scopies[p]:
          cp.wait()
      convert(inbufs[p], outbufs[p])
      t, c = divmod(step, _NCHUNK)
      dst = pl.ds(base + c * _CHUNK, _CHUNK)
      scopies[p] = tuple(
          pltpu.async_copy(outbufs[p], o.at[dst], ssems[p])
          for o in outs[t])
    for cps in scopies:
      for cp in cps:
        cp.wait()

  return emb_kernel


_emb_kernel = _make_kernel()


def kernel(inputs, W0, W1, W2):
  e0, e1, e2, e3, e4, e5 = _emb_kernel(inputs, W0, W1, W2)
  shape = (_B, _S, _DIM)
  return (e0.reshape(shape), e1.reshape(shape), e2.reshape(shape),
          e3.reshape(shape), e4.reshape(shape), e5.reshape(shape))
